# baseline (speedup 1.0000x reference)
import functools

import jax
import jax.numpy as jnp
from jax import lax
from jax.experimental import pallas as pl
from jax.experimental.pallas import tpu as pltpu
from jax.experimental.pallas import tpu_sc as plsc

N = 4096
L = 2
D = 128
ROWS = N * L
BI = 256
NB = N // BI
BM = 1024

_NC = 2
_NS = 16
_LANES = 16
_NW = _NC * _NS
_BPW = ROWS // _NW
_ICHUNK = 32


def _adj_reduce_body(adj_ref, rdeg_ref, istar_ref):
    b = pl.program_id(1)
    x = adj_ref[0]
    s = jnp.sum(x, axis=1, keepdims=True)
    rdeg_ref[0, pl.ds(b * BI, BI), :] = lax.rsqrt(s.astype(jnp.float32) + 1.0)
    ii = b * BI + lax.broadcasted_iota(jnp.int32, (BI, N), 0)
    cand = jnp.where(x == 1, ii, -1)
    cm = jnp.max(cand, axis=0, keepdims=True)

    @pl.when(b == 0)
    def _():
        istar_ref[0] = lax.broadcasted_iota(jnp.int32, (1, N), 1)

    istar_ref[0] = jnp.maximum(istar_ref[0], cm)


def _matmul_body(f_ref, w_ref, o_ref):
    o_ref[...] = lax.dot_general(
        f_ref[...], w_ref[...], (((1,), (1,)), ((), ())),
        preferred_element_type=jnp.float32)


def _sc_gather_scale(istar_hbm, rdeg_hbm, trans_hbm, rows_hbm, scale_hbm,
                     istar_v, rdeg_v, fidx_v, scale_v, rows_v, sem):
    wid = lax.axis_index("s") * _NC + lax.axis_index("c")
    base = wid * _BPW
    pltpu.sync_copy(istar_hbm, istar_v)
    pltpu.sync_copy(rdeg_hbm, rdeg_v)

    iot = lax.broadcasted_iota(jnp.int32, (_LANES,), 0)
    l_idx = iot & 1
    half = iot >> 1

    for k in range(_BPW // _LANES):
        j_idx = ((base >> 1) + 8 * k) + half
        flat_j = l_idx * N + j_idx
        ist = plsc.load_gather(istar_v, [flat_j])
        rs = plsc.load_gather(rdeg_v, [l_idx * N + ist])
        rd = plsc.load_gather(rdeg_v, [flat_j])
        scale_v[pl.ds(_LANES * k, _LANES)] = rs * rd
        fidx_v[(_LANES * k) // _ICHUNK,
               pl.ds((_LANES * k) % _ICHUNK, _LANES)] = ist * 2 + l_idx

    copies = [
        pltpu.async_copy(trans_hbm.at[fidx_v.at[t]],
                         rows_v.at[pl.ds(t * _ICHUNK, _ICHUNK)], sem)
        for t in range(_BPW // _ICHUNK)
    ]
    for cp in copies:
        cp.wait()

    pltpu.sync_copy(rows_v, rows_hbm.at[pl.ds(base, _BPW)])
    pltpu.sync_copy(scale_v, scale_hbm.at[pl.ds(base, _BPW)])


def _scale_body(r_ref, s_ref, o_ref):
    o_ref[...] = r_ref[...] * s_ref[...]


@functools.cache
def _sc_kernel():
    mesh = plsc.VectorSubcoreMesh(core_axis_name="c", subcore_axis_name="s",
                                  num_cores=_NC, num_subcores=_NS)
    return pl.kernel(
        _sc_gather_scale,
        out_type=(jax.ShapeDtypeStruct((ROWS, D), jnp.float32),
                  jax.ShapeDtypeStruct((ROWS,), jnp.float32)),
        mesh=mesh,
        scratch_types=[
            pltpu.VMEM((ROWS,), jnp.int32),
            pltpu.VMEM((ROWS,), jnp.float32),
            pltpu.VMEM((_BPW // _ICHUNK, _ICHUNK), jnp.int32),
            pltpu.VMEM((_BPW,), jnp.float32),
            pltpu.VMEM((_BPW, D), jnp.float32),
            pltpu.SemaphoreType.DMA,
        ],
        compiler_params=pltpu.CompilerParams(needs_layout_passes=False),
    )


def kernel(feature, adj, W):
    rdeg, istar = pl.pallas_call(
        _adj_reduce_body,
        grid=(L, NB),
        in_specs=[pl.BlockSpec((1, BI, N), lambda l, b: (l, b, 0))],
        out_specs=[pl.BlockSpec((1, N, 1), lambda l, b: (l, 0, 0)),
                   pl.BlockSpec((1, 1, N), lambda l, b: (l, 0, 0))],
        out_shape=[jax.ShapeDtypeStruct((L, N, 1), jnp.float32),
                   jax.ShapeDtypeStruct((L, 1, N), jnp.int32)],
    )(adj)

    trans = pl.pallas_call(
        _matmul_body,
        grid=(ROWS // BM,),
        in_specs=[pl.BlockSpec((BM, D), lambda m: (m, 0)),
                  pl.BlockSpec((D, D), lambda m: (0, 0))],
        out_specs=pl.BlockSpec((BM, D), lambda m: (m, 0)),
        out_shape=jax.ShapeDtypeStruct((ROWS, D), jnp.float32),
    )(feature.reshape(ROWS, D), W)

    rows, scale = _sc_kernel()(istar.reshape(ROWS), rdeg.reshape(ROWS), trans)

    out2 = pl.pallas_call(
        _scale_body,
        grid=(ROWS // BM,),
        in_specs=[pl.BlockSpec((BM, D), lambda m: (m, 0)),
                  pl.BlockSpec((BM, 1), lambda m: (m, 0))],
        out_specs=pl.BlockSpec((BM, D), lambda m: (m, 0)),
        out_shape=jax.ShapeDtypeStruct((ROWS, D), jnp.float32),
    )(rows, scale.reshape(ROWS, 1))
    return out2.reshape(N, L, D)


# speedup vs baseline: 1.8614x; 1.8614x over previous
import functools

import jax
import jax.numpy as jnp
from jax import lax
from jax.experimental import pallas as pl
from jax.experimental.pallas import tpu as pltpu
from jax.experimental.pallas import tpu_sc as plsc

N = 4096
L = 2
D = 128
ROWS = N * L
BI = 256
NB = N // BI
BM = 1024

_NC = 2
_NS = 16
_LANES = 16
_NW = _NC * _NS
_BPW = ROWS // _NW
_ICHUNK = 32
_SHARE = ROWS // _NS


def _adj_reduce_body(adj_ref, rdeg_ref, istar_ref):
    b = pl.program_id(1)
    x = adj_ref[0]
    s = jnp.sum(x, axis=1, keepdims=True)
    rdeg_ref[0, pl.ds(b * BI, BI), :] = lax.rsqrt(s.astype(jnp.float32) + 1.0)
    ii = b * BI + lax.broadcasted_iota(jnp.int32, (BI, N), 0)
    cand = jnp.where(x == 1, ii, -1)
    cm = jnp.max(cand, axis=0, keepdims=True)

    @pl.when(b == 0)
    def _():
        istar_ref[0] = lax.broadcasted_iota(jnp.int32, (1, N), 1)

    istar_ref[0] = jnp.maximum(istar_ref[0], cm)


def _epilogue_body(r_ref, w_ref, s_ref, o_ref):
    t = lax.dot_general(r_ref[...], w_ref[...], (((1,), (1,)), ((), ())),
                        preferred_element_type=jnp.float32)
    o_ref[...] = t * s_ref[...]


def _sc_gather_scale(istar_hbm, rdeg_hbm, feat_hbm, rows_hbm, scale_hbm,
                     istar_v, rdeg_v, fidx_v, scale_v, rows_v, table_s, sem):
    sid = lax.axis_index("s")
    wid = sid * _NC + lax.axis_index("c")
    base = wid * _BPW

    pltpu.sync_copy(feat_hbm.at[pl.ds(sid * _SHARE, _SHARE)],
                    table_s.at[pl.ds(sid * _SHARE, _SHARE)])
    pltpu.sync_copy(istar_hbm, istar_v)
    pltpu.sync_copy(rdeg_hbm, rdeg_v)
    plsc.subcore_barrier()

    iot = lax.broadcasted_iota(jnp.int32, (_LANES,), 0)
    l_idx = iot & 1
    half = iot >> 1

    for k in range(_BPW // _LANES):
        j_idx = ((base >> 1) + 8 * k) + half
        flat_j = l_idx * N + j_idx
        ist = plsc.load_gather(istar_v, [flat_j])
        rs = plsc.load_gather(rdeg_v, [l_idx * N + ist])
        rd = plsc.load_gather(rdeg_v, [flat_j])
        scale_v[pl.ds(_LANES * k, _LANES)] = rs * rd
        fidx_v[(_LANES * k) // _ICHUNK,
               pl.ds((_LANES * k) % _ICHUNK, _LANES)] = ist * 2 + l_idx

    copies = [
        pltpu.async_copy(table_s.at[fidx_v.at[t]],
                         rows_v.at[pl.ds(t * _ICHUNK, _ICHUNK)], sem)
        for t in range(_BPW // _ICHUNK)
    ]
    for cp in copies:
        cp.wait()

    pltpu.sync_copy(rows_v, rows_hbm.at[pl.ds(base, _BPW)])
    pltpu.sync_copy(scale_v, scale_hbm.at[pl.ds(base, _BPW)])


@functools.cache
def _sc_kernel():
    mesh = plsc.VectorSubcoreMesh(core_axis_name="c", subcore_axis_name="s",
                                  num_cores=_NC, num_subcores=_NS)
    return pl.kernel(
        _sc_gather_scale,
        out_type=(jax.ShapeDtypeStruct((ROWS, D), jnp.float32),
                  jax.ShapeDtypeStruct((ROWS,), jnp.float32)),
        mesh=mesh,
        scratch_types=[
            pltpu.VMEM((ROWS,), jnp.int32),
            pltpu.VMEM((ROWS,), jnp.float32),
            pltpu.VMEM((_BPW // _ICHUNK, _ICHUNK), jnp.int32),
            pltpu.VMEM((_BPW,), jnp.float32),
            pltpu.VMEM((_BPW, D), jnp.float32),
            pltpu.VMEM_SHARED((ROWS, D), jnp.float32),
            pltpu.SemaphoreType.DMA,
        ],
        compiler_params=pltpu.CompilerParams(needs_layout_passes=False),
    )


def kernel(feature, adj, W):
    rdeg, istar = pl.pallas_call(
        _adj_reduce_body,
        grid=(L, NB),
        in_specs=[pl.BlockSpec((1, BI, N), lambda l, b: (l, b, 0))],
        out_specs=[pl.BlockSpec((1, N, 1), lambda l, b: (l, 0, 0)),
                   pl.BlockSpec((1, 1, N), lambda l, b: (l, 0, 0))],
        out_shape=[jax.ShapeDtypeStruct((L, N, 1), jnp.float32),
                   jax.ShapeDtypeStruct((L, 1, N), jnp.int32)],
    )(adj)

    feat2 = feature.reshape(ROWS, D)
    rows, scale = _sc_kernel()(istar.reshape(ROWS), rdeg.reshape(ROWS), feat2)

    out2 = pl.pallas_call(
        _epilogue_body,
        grid=(ROWS // BM,),
        in_specs=[pl.BlockSpec((BM, D), lambda m: (m, 0)),
                  pl.BlockSpec((D, D), lambda m: (0, 0)),
                  pl.BlockSpec((BM, 1), lambda m: (m, 0))],
        out_specs=pl.BlockSpec((BM, D), lambda m: (m, 0)),
        out_shape=jax.ShapeDtypeStruct((ROWS, D), jnp.float32),
    )(rows, W, scale.reshape(ROWS, 1))
    return out2.reshape(N, L, D)


# speedup vs baseline: 2.0147x; 1.0824x over previous
import functools

import jax
import jax.numpy as jnp
from jax import lax
from jax.experimental import pallas as pl
from jax.experimental.pallas import tpu as pltpu
from jax.experimental.pallas import tpu_sc as plsc

N = 4096
L = 2
D = 128
ROWS = N * L
BI = 256
NB = N // BI
BM = 1024

_NC = 2
_NS = 16
_LANES = 16
_NW = _NC * _NS
_BPW = ROWS // _NW
_ICHUNK = 32
_SHARE = ROWS // _NS


def _adj_reduce_body(adj_ref, rdeg_ref, istar_ref):
    b = pl.program_id(1)
    x = adj_ref[0]
    s = jnp.sum(x, axis=1)
    rdeg_ref[0, 0, pl.ds(b * BI, BI)] = lax.rsqrt(s.astype(jnp.float32) + 1.0)
    ii = b * BI + lax.broadcasted_iota(jnp.int32, (BI, N), 0)
    cand = jnp.where(x == 1, ii, -1)
    cm = jnp.max(cand, axis=0, keepdims=True)

    @pl.when(b == 0)
    def _():
        istar_ref[0] = lax.broadcasted_iota(jnp.int32, (1, N), 1)

    istar_ref[0] = jnp.maximum(istar_ref[0], cm)


def _epilogue_body(r_ref, w_ref, s_ref, o_ref):
    t = lax.dot_general(r_ref[...], w_ref[...], (((1,), (1,)), ((), ())),
                        preferred_element_type=jnp.float32)
    o_ref[...] = t * s_ref[...].reshape(BM, 1)


def _sc_gather_scale(istar_hbm, rdeg_hbm, feat_hbm, rows_hbm, scale_hbm,
                     istar_v, rdeg_v, fidx_v, scale_v, rows_v, table_s, sem):
    sid = lax.axis_index("s")
    wid = sid * _NC + lax.axis_index("c")
    base = wid * _BPW

    pltpu.sync_copy(feat_hbm.at[pl.ds(sid * _SHARE, _SHARE)],
                    table_s.at[pl.ds(sid * _SHARE, _SHARE)])
    pltpu.sync_copy(istar_hbm, istar_v)
    pltpu.sync_copy(rdeg_hbm, rdeg_v)
    plsc.subcore_barrier()

    iot = lax.broadcasted_iota(jnp.int32, (_LANES,), 0)
    l_idx = iot & 1
    half = iot >> 1

    for k in range(_BPW // _LANES):
        j_idx = ((base >> 1) + 8 * k) + half
        flat_j = l_idx * N + j_idx
        ist = plsc.load_gather(istar_v, [flat_j])
        rs = plsc.load_gather(rdeg_v, [l_idx * N + ist])
        rd = plsc.load_gather(rdeg_v, [flat_j])
        scale_v[pl.ds(_LANES * k, _LANES)] = rs * rd
        fidx_v[(_LANES * k) // _ICHUNK,
               pl.ds((_LANES * k) % _ICHUNK, _LANES)] = ist * 2 + l_idx

    copies = [
        pltpu.async_copy(table_s.at[fidx_v.at[t]],
                         rows_v.at[pl.ds(t * _ICHUNK, _ICHUNK)], sem)
        for t in range(_BPW // _ICHUNK)
    ]
    for cp in copies:
        cp.wait()

    pltpu.sync_copy(rows_v, rows_hbm.at[pl.ds(base, _BPW)])
    pltpu.sync_copy(scale_v, scale_hbm.at[pl.ds(base, _BPW)])


@functools.cache
def _sc_kernel():
    mesh = plsc.VectorSubcoreMesh(core_axis_name="c", subcore_axis_name="s",
                                  num_cores=_NC, num_subcores=_NS)
    return pl.kernel(
        _sc_gather_scale,
        out_type=(jax.ShapeDtypeStruct((ROWS, D), jnp.float32),
                  jax.ShapeDtypeStruct((ROWS,), jnp.float32)),
        mesh=mesh,
        scratch_types=[
            pltpu.VMEM((ROWS,), jnp.int32),
            pltpu.VMEM((ROWS,), jnp.float32),
            pltpu.VMEM((_BPW // _ICHUNK, _ICHUNK), jnp.int32),
            pltpu.VMEM((_BPW,), jnp.float32),
            pltpu.VMEM((_BPW, D), jnp.float32),
            pltpu.VMEM_SHARED((ROWS, D), jnp.float32),
            pltpu.SemaphoreType.DMA,
        ],
        compiler_params=pltpu.CompilerParams(needs_layout_passes=False),
    )


def kernel(feature, adj, W):
    rdeg, istar = pl.pallas_call(
        _adj_reduce_body,
        grid=(L, NB),
        in_specs=[pl.BlockSpec((1, BI, N), lambda l, b: (l, b, 0))],
        out_specs=[pl.BlockSpec((1, 1, N), lambda l, b: (l, 0, 0)),
                   pl.BlockSpec((1, 1, N), lambda l, b: (l, 0, 0))],
        out_shape=[jax.ShapeDtypeStruct((L, 1, N), jnp.float32),
                   jax.ShapeDtypeStruct((L, 1, N), jnp.int32)],
    )(adj)

    feat2 = feature.reshape(ROWS, D)
    rows, scale = _sc_kernel()(istar.reshape(ROWS), rdeg.reshape(ROWS), feat2)

    out2 = pl.pallas_call(
        _epilogue_body,
        grid=(ROWS // BM,),
        in_specs=[pl.BlockSpec((BM, D), lambda m: (m, 0)),
                  pl.BlockSpec((D, D), lambda m: (0, 0)),
                  pl.BlockSpec((BM,), lambda m: (m,))],
        out_specs=pl.BlockSpec((BM, D), lambda m: (m, 0)),
        out_shape=jax.ShapeDtypeStruct((ROWS, D), jnp.float32),
    )(rows, W, scale)
    return out2.reshape(N, L, D)


# speedup vs baseline: 2.2237x; 1.1037x over previous
import functools

import jax
import jax.numpy as jnp
from jax import lax
from jax.experimental import pallas as pl
from jax.experimental.pallas import tpu as pltpu
from jax.experimental.pallas import tpu_sc as plsc

N = 4096
L = 2
D = 128
ROWS = N * L
BI = 512
NB = N // BI
BM = 1024

_NC = 2
_NS = 16
_LANES = 16
_NW = _NC * _NS
_BPW = ROWS // _NW
_ICHUNK = 32
_SHARE = ROWS // _NS


def _adj_reduce_body(adj_ref, rdeg_ref, istar_ref):
    b = pl.program_id(1)
    x = adj_ref[0]
    s = jnp.sum(x, axis=1)
    rdeg_ref[0, 0, pl.ds(b * BI, BI)] = lax.rsqrt(s.astype(jnp.float32) + 1.0)
    ii = b * BI + lax.broadcasted_iota(jnp.int32, (BI, N), 0)
    cand = jnp.where(x == 1, ii, -1)
    cm = jnp.max(cand, axis=0, keepdims=True)

    @pl.when(b == 0)
    def _():
        istar_ref[0] = lax.broadcasted_iota(jnp.int32, (1, N), 1)

    istar_ref[0] = jnp.maximum(istar_ref[0], cm)


def _epilogue_body(r_ref, w_ref, s_ref, o_ref):
    t = lax.dot_general(r_ref[...], w_ref[...], (((1,), (1,)), ((), ())),
                        preferred_element_type=jnp.float32)
    o_ref[...] = t * s_ref[...].reshape(BM, 1)


def _sc_gather_scale(istar_hbm, rdeg_hbm, feat_hbm, rows_hbm, scale_hbm,
                     istar_v, rdeg_v, fidx_v, scale_v, rows_v, table_s, sem):
    sid = lax.axis_index("s")
    wid = sid * _NC + lax.axis_index("c")
    base = wid * _BPW

    pltpu.sync_copy(feat_hbm.at[pl.ds(sid * _SHARE, _SHARE)],
                    table_s.at[pl.ds(sid * _SHARE, _SHARE)])
    pltpu.sync_copy(istar_hbm, istar_v)
    pltpu.sync_copy(rdeg_hbm, rdeg_v)
    plsc.subcore_barrier()

    iot = lax.broadcasted_iota(jnp.int32, (_LANES,), 0)
    l_idx = iot & 1
    half = iot >> 1

    for k in range(_BPW // _LANES):
        j_idx = ((base >> 1) + 8 * k) + half
        flat_j = l_idx * N + j_idx
        ist = plsc.load_gather(istar_v, [flat_j])
        rs = plsc.load_gather(rdeg_v, [l_idx * N + ist])
        rd = plsc.load_gather(rdeg_v, [flat_j])
        scale_v[pl.ds(_LANES * k, _LANES)] = rs * rd
        fidx_v[(_LANES * k) // _ICHUNK,
               pl.ds((_LANES * k) % _ICHUNK, _LANES)] = ist * 2 + l_idx

    copies = [
        pltpu.async_copy(table_s.at[fidx_v.at[t]],
                         rows_v.at[pl.ds(t * _ICHUNK, _ICHUNK)], sem)
        for t in range(_BPW // _ICHUNK)
    ]
    for cp in copies:
        cp.wait()

    pltpu.sync_copy(rows_v, rows_hbm.at[pl.ds(base, _BPW)])
    pltpu.sync_copy(scale_v, scale_hbm.at[pl.ds(base, _BPW)])


@functools.cache
def _sc_kernel():
    mesh = plsc.VectorSubcoreMesh(core_axis_name="c", subcore_axis_name="s",
                                  num_cores=_NC, num_subcores=_NS)
    return pl.kernel(
        _sc_gather_scale,
        out_type=(jax.ShapeDtypeStruct((ROWS, D), jnp.float32),
                  jax.ShapeDtypeStruct((ROWS,), jnp.float32)),
        mesh=mesh,
        scratch_types=[
            pltpu.VMEM((ROWS,), jnp.int32),
            pltpu.VMEM((ROWS,), jnp.float32),
            pltpu.VMEM((_BPW // _ICHUNK, _ICHUNK), jnp.int32),
            pltpu.VMEM((_BPW,), jnp.float32),
            pltpu.VMEM((_BPW, D), jnp.float32),
            pltpu.VMEM_SHARED((ROWS, D), jnp.float32),
            pltpu.SemaphoreType.DMA,
        ],
        compiler_params=pltpu.CompilerParams(needs_layout_passes=False),
    )


def kernel(feature, adj, W):
    rdeg, istar = pl.pallas_call(
        _adj_reduce_body,
        grid=(L, NB),
        in_specs=[pl.BlockSpec((1, BI, N), lambda l, b: (l, b, 0))],
        out_specs=[pl.BlockSpec((1, 1, N), lambda l, b: (l, 0, 0)),
                   pl.BlockSpec((1, 1, N), lambda l, b: (l, 0, 0))],
        out_shape=[jax.ShapeDtypeStruct((L, 1, N), jnp.float32),
                   jax.ShapeDtypeStruct((L, 1, N), jnp.int32)],
    )(adj)

    feat2 = feature.reshape(ROWS, D)
    rows, scale = _sc_kernel()(istar.reshape(ROWS), rdeg.reshape(ROWS), feat2)

    out2 = pl.pallas_call(
        _epilogue_body,
        grid=(ROWS // BM,),
        in_specs=[pl.BlockSpec((BM, D), lambda m: (m, 0)),
                  pl.BlockSpec((D, D), lambda m: (0, 0)),
                  pl.BlockSpec((BM,), lambda m: (m,))],
        out_specs=pl.BlockSpec((BM, D), lambda m: (m, 0)),
        out_shape=jax.ShapeDtypeStruct((ROWS, D), jnp.float32),
    )(rows, W, scale)
    return out2.reshape(N, L, D)


# speedup vs baseline: 2.2981x; 1.0334x over previous
import functools

import jax
import jax.numpy as jnp
from jax import lax
from jax.experimental import pallas as pl
from jax.experimental.pallas import tpu as pltpu
from jax.experimental.pallas import tpu_sc as plsc

N = 4096
L = 2
D = 128
ROWS = N * L
BI = 1024
NB = N // BI
BM = 1024

_NC = 2
_NS = 16
_LANES = 16
_NW = _NC * _NS
_BPW = ROWS // _NW
_ICHUNK = 32
_SHARE = ROWS // _NS


def _adj_reduce_body(adj_ref, rdeg_ref, istar_ref):
    b = pl.program_id(1)
    x = adj_ref[0]
    s = jnp.sum(x, axis=1)
    rdeg_ref[0, 0, pl.ds(b * BI, BI)] = lax.rsqrt(s.astype(jnp.float32) + 1.0)
    ii = b * BI + lax.broadcasted_iota(jnp.int32, (BI, N), 0)
    cand = jnp.where(x == 1, ii, -1)
    cm = jnp.max(cand, axis=0, keepdims=True)

    @pl.when(b == 0)
    def _():
        istar_ref[0] = lax.broadcasted_iota(jnp.int32, (1, N), 1)

    istar_ref[0] = jnp.maximum(istar_ref[0], cm)


def _epilogue_body(r_ref, w_ref, s_ref, o_ref):
    t = lax.dot_general(r_ref[...], w_ref[...], (((1,), (1,)), ((), ())),
                        preferred_element_type=jnp.float32)
    o_ref[...] = t * s_ref[...].reshape(BM, 1)


def _sc_gather_scale(istar_hbm, rdeg_hbm, feat_hbm, rows_hbm, scale_hbm,
                     istar_v, rdeg_v, fidx_v, scale_v, rows_v, table_s, sem):
    sid = lax.axis_index("s")
    wid = sid * _NC + lax.axis_index("c")
    base = wid * _BPW

    pltpu.sync_copy(feat_hbm.at[pl.ds(sid * _SHARE, _SHARE)],
                    table_s.at[pl.ds(sid * _SHARE, _SHARE)])
    pltpu.sync_copy(istar_hbm, istar_v)
    pltpu.sync_copy(rdeg_hbm, rdeg_v)
    plsc.subcore_barrier()

    iot = lax.broadcasted_iota(jnp.int32, (_LANES,), 0)
    l_idx = iot & 1
    half = iot >> 1

    for k in range(_BPW // _LANES):
        j_idx = ((base >> 1) + 8 * k) + half
        flat_j = l_idx * N + j_idx
        ist = plsc.load_gather(istar_v, [flat_j])
        rs = plsc.load_gather(rdeg_v, [l_idx * N + ist])
        rd = plsc.load_gather(rdeg_v, [flat_j])
        scale_v[pl.ds(_LANES * k, _LANES)] = rs * rd
        fidx_v[(_LANES * k) // _ICHUNK,
               pl.ds((_LANES * k) % _ICHUNK, _LANES)] = ist * 2 + l_idx

    copies = [
        pltpu.async_copy(table_s.at[fidx_v.at[t]],
                         rows_v.at[pl.ds(t * _ICHUNK, _ICHUNK)], sem)
        for t in range(_BPW // _ICHUNK)
    ]
    for cp in copies:
        cp.wait()

    pltpu.sync_copy(rows_v, rows_hbm.at[pl.ds(base, _BPW)])
    pltpu.sync_copy(scale_v, scale_hbm.at[pl.ds(base, _BPW)])


@functools.cache
def _sc_kernel():
    mesh = plsc.VectorSubcoreMesh(core_axis_name="c", subcore_axis_name="s",
                                  num_cores=_NC, num_subcores=_NS)
    return pl.kernel(
        _sc_gather_scale,
        out_type=(jax.ShapeDtypeStruct((ROWS, D), jnp.float32),
                  jax.ShapeDtypeStruct((ROWS,), jnp.float32)),
        mesh=mesh,
        scratch_types=[
            pltpu.VMEM((ROWS,), jnp.int32),
            pltpu.VMEM((ROWS,), jnp.float32),
            pltpu.VMEM((_BPW // _ICHUNK, _ICHUNK), jnp.int32),
            pltpu.VMEM((_BPW,), jnp.float32),
            pltpu.VMEM((_BPW, D), jnp.float32),
            pltpu.VMEM_SHARED((ROWS, D), jnp.float32),
            pltpu.SemaphoreType.DMA,
        ],
        compiler_params=pltpu.CompilerParams(needs_layout_passes=False),
    )


def kernel(feature, adj, W):
    rdeg, istar = pl.pallas_call(
        _adj_reduce_body,
        grid=(L, NB),
        in_specs=[pl.BlockSpec((1, BI, N), lambda l, b: (l, b, 0))],
        out_specs=[pl.BlockSpec((1, 1, N), lambda l, b: (l, 0, 0)),
                   pl.BlockSpec((1, 1, N), lambda l, b: (l, 0, 0))],
        out_shape=[jax.ShapeDtypeStruct((L, 1, N), jnp.float32),
                   jax.ShapeDtypeStruct((L, 1, N), jnp.int32)],
    )(adj)

    feat2 = feature.reshape(ROWS, D)
    rows, scale = _sc_kernel()(istar.reshape(ROWS), rdeg.reshape(ROWS), feat2)

    out2 = pl.pallas_call(
        _epilogue_body,
        grid=(ROWS // BM,),
        in_specs=[pl.BlockSpec((BM, D), lambda m: (m, 0)),
                  pl.BlockSpec((D, D), lambda m: (0, 0)),
                  pl.BlockSpec((BM,), lambda m: (m,))],
        out_specs=pl.BlockSpec((BM, D), lambda m: (m, 0)),
        out_shape=jax.ShapeDtypeStruct((ROWS, D), jnp.float32),
    )(rows, W, scale)
    return out2.reshape(N, L, D)


# speedup vs baseline: 2.3148x; 1.0073x over previous
import functools

import jax
import jax.numpy as jnp
from jax import lax
from jax.experimental import pallas as pl
from jax.experimental.pallas import tpu as pltpu
from jax.experimental.pallas import tpu_sc as plsc

N = 4096
L = 2
D = 128
ROWS = N * L
BI = 1024
NB = N // BI
BM = 1024

_NC = 2
_NS = 16
_LANES = 16
_NW = _NC * _NS
_BPW = ROWS // _NW
_ICHUNK = 32
_SHARE = ROWS // _NS


def _adj_reduce_body(adj_ref, rdeg_ref, istar_ref):
    b = pl.program_id(1)
    x = adj_ref[0]
    s = jnp.sum(x, axis=1)
    rdeg_ref[0, 0, pl.ds(b * BI, BI)] = lax.rsqrt(s.astype(jnp.float32) + 1.0)
    ii = b * BI + lax.broadcasted_iota(jnp.int32, (BI, N), 0)
    cand = jnp.where(x == 1, ii, -1)
    cm = jnp.max(cand, axis=0, keepdims=True)

    @pl.when(b == 0)
    def _():
        istar_ref[0] = lax.broadcasted_iota(jnp.int32, (1, N), 1)

    istar_ref[0] = jnp.maximum(istar_ref[0], cm)


def _epilogue_body(r_ref, w_ref, s_ref, o_ref):
    t = lax.dot_general(r_ref[...], w_ref[...], (((1,), (1,)), ((), ())),
                        preferred_element_type=jnp.float32)
    o_ref[...] = t * s_ref[...].reshape(BM, 1)


def _sc_gather_scale(istar_hbm, rdeg_hbm, feat_hbm, rows_hbm, scale_hbm,
                     istar_v, rdeg_v, fidx_v, scale_v, rows_v, table_s, sem):
    sid = lax.axis_index("s")
    wid = sid * _NC + lax.axis_index("c")
    base = wid * _BPW

    stage = pltpu.async_copy(feat_hbm.at[pl.ds(sid * _SHARE, _SHARE)],
                             table_s.at[pl.ds(sid * _SHARE, _SHARE)], sem)
    pltpu.sync_copy(istar_hbm, istar_v)
    pltpu.sync_copy(rdeg_hbm, rdeg_v)

    iot = lax.broadcasted_iota(jnp.int32, (_LANES,), 0)
    l_idx = iot & 1
    half = iot >> 1

    def _index_step(k, carry):
        j_idx = ((base >> 1) + 8 * k) + half
        flat_j = l_idx * N + j_idx
        ist = plsc.load_gather(istar_v, [flat_j])
        rs = plsc.load_gather(rdeg_v, [l_idx * N + ist])
        rd = plsc.load_gather(rdeg_v, [flat_j])
        scale_v[pl.ds(_LANES * k, _LANES)] = rs * rd
        fidx_v[pl.ds(_LANES * k, _LANES)] = ist * 2 + l_idx
        return carry

    lax.fori_loop(0, _BPW // _LANES, _index_step, 0)

    stage.wait()
    plsc.subcore_barrier()

    copies = [
        pltpu.async_copy(table_s.at[fidx_v.at[pl.ds(t * _ICHUNK, _ICHUNK)]],
                         rows_v.at[pl.ds(t * _ICHUNK, _ICHUNK)], sem)
        for t in range(_BPW // _ICHUNK)
    ]
    for cp in copies:
        cp.wait()

    pltpu.sync_copy(rows_v, rows_hbm.at[pl.ds(base, _BPW)])
    pltpu.sync_copy(scale_v, scale_hbm.at[pl.ds(base, _BPW)])


@functools.cache
def _sc_kernel():
    mesh = plsc.VectorSubcoreMesh(core_axis_name="c", subcore_axis_name="s",
                                  num_cores=_NC, num_subcores=_NS)
    return pl.kernel(
        _sc_gather_scale,
        out_type=(jax.ShapeDtypeStruct((ROWS, D), jnp.float32),
                  jax.ShapeDtypeStruct((ROWS,), jnp.float32)),
        mesh=mesh,
        scratch_types=[
            pltpu.VMEM((ROWS,), jnp.int32),
            pltpu.VMEM((ROWS,), jnp.float32),
            pltpu.VMEM((_BPW,), jnp.int32),
            pltpu.VMEM((_BPW,), jnp.float32),
            pltpu.VMEM((_BPW, D), jnp.float32),
            pltpu.VMEM_SHARED((ROWS, D), jnp.float32),
            pltpu.SemaphoreType.DMA,
        ],
        compiler_params=pltpu.CompilerParams(needs_layout_passes=False),
    )


def kernel(feature, adj, W):
    rdeg, istar = pl.pallas_call(
        _adj_reduce_body,
        grid=(L, NB),
        in_specs=[pl.BlockSpec((1, BI, N), lambda l, b: (l, b, 0))],
        out_specs=[pl.BlockSpec((1, 1, N), lambda l, b: (l, 0, 0)),
                   pl.BlockSpec((1, 1, N), lambda l, b: (l, 0, 0))],
        out_shape=[jax.ShapeDtypeStruct((L, 1, N), jnp.float32),
                   jax.ShapeDtypeStruct((L, 1, N), jnp.int32)],
    )(adj)

    feat2 = feature.reshape(ROWS, D)
    rows, scale = _sc_kernel()(istar.reshape(ROWS), rdeg.reshape(ROWS), feat2)

    out2 = pl.pallas_call(
        _epilogue_body,
        grid=(ROWS // BM,),
        in_specs=[pl.BlockSpec((BM, D), lambda m: (m, 0)),
                  pl.BlockSpec((D, D), lambda m: (0, 0)),
                  pl.BlockSpec((BM,), lambda m: (m,))],
        out_specs=pl.BlockSpec((BM, D), lambda m: (m, 0)),
        out_shape=jax.ShapeDtypeStruct((ROWS, D), jnp.float32),
    )(rows, W, scale)
    return out2.reshape(N, L, D)


# speedup vs baseline: 2.3805x; 1.0284x over previous
import functools

import jax
import jax.numpy as jnp
from jax import lax
from jax.experimental import pallas as pl
from jax.experimental.pallas import tpu as pltpu
from jax.experimental.pallas import tpu_sc as plsc

N = 4096
L = 2
D = 128
ROWS = N * L
BI = 1024
NB = N // BI
BM = 2048

_NC = 2
_NS = 16
_LANES = 16
_NW = _NC * _NS
_BPW = ROWS // _NW
_ICHUNK = 32
_SHARE = ROWS // _NS


def _adj_reduce_body(adj_ref, rdeg_ref, istar_ref):
    b = pl.program_id(1)
    x = adj_ref[0]
    s = jnp.sum(x, axis=1)
    rdeg_ref[0, 0, pl.ds(b * BI, BI)] = lax.rsqrt(s.astype(jnp.float32) + 1.0)
    ii = b * BI + lax.broadcasted_iota(jnp.int32, (BI, N), 0)
    cand = jnp.where(x == 1, ii, -1)
    cm = jnp.max(cand, axis=0, keepdims=True)

    @pl.when(b == 0)
    def _():
        istar_ref[0] = lax.broadcasted_iota(jnp.int32, (1, N), 1)

    istar_ref[0] = jnp.maximum(istar_ref[0], cm)


def _epilogue_body(r_ref, w_ref, s_ref, o_ref):
    t = lax.dot_general(r_ref[...], w_ref[...], (((1,), (1,)), ((), ())),
                        preferred_element_type=jnp.float32)
    o_ref[...] = t * s_ref[...].reshape(BM, 1)


def _sc_gather_scale(istar_hbm, rdeg_hbm, feat_hbm, rows_hbm, scale_hbm,
                     istar_v, rdeg_v, fidx_v, scale_v, rows_v, table_s, sem):
    sid = lax.axis_index("s")
    wid = sid * _NC + lax.axis_index("c")
    base = wid * _BPW

    stage = pltpu.async_copy(feat_hbm.at[pl.ds(sid * _SHARE, _SHARE)],
                             table_s.at[pl.ds(sid * _SHARE, _SHARE)], sem)
    pltpu.sync_copy(istar_hbm, istar_v)
    pltpu.sync_copy(rdeg_hbm, rdeg_v)

    iot = lax.broadcasted_iota(jnp.int32, (_LANES,), 0)
    l_idx = iot & 1
    half = iot >> 1

    def _index_step(k, carry):
        j_idx = ((base >> 1) + 8 * k) + half
        flat_j = l_idx * N + j_idx
        ist = plsc.load_gather(istar_v, [flat_j])
        rs = plsc.load_gather(rdeg_v, [l_idx * N + ist])
        rd = plsc.load_gather(rdeg_v, [flat_j])
        scale_v[pl.ds(_LANES * k, _LANES)] = rs * rd
        fidx_v[pl.ds(_LANES * k, _LANES)] = ist * 2 + l_idx
        return carry

    lax.fori_loop(0, _BPW // _LANES, _index_step, 0)

    stage.wait()
    plsc.subcore_barrier()

    copies = [
        pltpu.async_copy(table_s.at[fidx_v.at[pl.ds(t * _ICHUNK, _ICHUNK)]],
                         rows_v.at[pl.ds(t * _ICHUNK, _ICHUNK)], sem)
        for t in range(_BPW // _ICHUNK)
    ]
    for cp in copies:
        cp.wait()

    pltpu.sync_copy(rows_v, rows_hbm.at[pl.ds(base, _BPW)])
    pltpu.sync_copy(scale_v, scale_hbm.at[pl.ds(base, _BPW)])


@functools.cache
def _sc_kernel():
    mesh = plsc.VectorSubcoreMesh(core_axis_name="c", subcore_axis_name="s",
                                  num_cores=_NC, num_subcores=_NS)
    return pl.kernel(
        _sc_gather_scale,
        out_type=(jax.ShapeDtypeStruct((ROWS, D), jnp.float32),
                  jax.ShapeDtypeStruct((ROWS,), jnp.float32)),
        mesh=mesh,
        scratch_types=[
            pltpu.VMEM((ROWS,), jnp.int32),
            pltpu.VMEM((ROWS,), jnp.float32),
            pltpu.VMEM((_BPW,), jnp.int32),
            pltpu.VMEM((_BPW,), jnp.float32),
            pltpu.VMEM((_BPW, D), jnp.float32),
            pltpu.VMEM_SHARED((ROWS, D), jnp.float32),
            pltpu.SemaphoreType.DMA,
        ],
        compiler_params=pltpu.CompilerParams(needs_layout_passes=False),
    )


def kernel(feature, adj, W):
    rdeg, istar = pl.pallas_call(
        _adj_reduce_body,
        grid=(L, NB),
        in_specs=[pl.BlockSpec((1, BI, N), lambda l, b: (l, b, 0))],
        out_specs=[pl.BlockSpec((1, 1, N), lambda l, b: (l, 0, 0)),
                   pl.BlockSpec((1, 1, N), lambda l, b: (l, 0, 0))],
        out_shape=[jax.ShapeDtypeStruct((L, 1, N), jnp.float32),
                   jax.ShapeDtypeStruct((L, 1, N), jnp.int32)],
    )(adj)

    feat2 = feature.reshape(ROWS, D)
    rows, scale = _sc_kernel()(istar.reshape(ROWS), rdeg.reshape(ROWS), feat2)

    out2 = pl.pallas_call(
        _epilogue_body,
        grid=(ROWS // BM,),
        in_specs=[pl.BlockSpec((BM, D), lambda m: (m, 0)),
                  pl.BlockSpec((D, D), lambda m: (0, 0)),
                  pl.BlockSpec((BM,), lambda m: (m,))],
        out_specs=pl.BlockSpec((BM, D), lambda m: (m, 0)),
        out_shape=jax.ShapeDtypeStruct((ROWS, D), jnp.float32),
    )(rows, W, scale)
    return out2.reshape(N, L, D)


# speedup vs baseline: 2.4189x; 1.0162x over previous
import functools

import jax
import jax.numpy as jnp
from jax import lax
from jax.experimental import pallas as pl
from jax.experimental.pallas import tpu as pltpu
from jax.experimental.pallas import tpu_sc as plsc

N = 4096
L = 2
D = 128
ROWS = N * L
BI = 1024
NB = N // BI
BM = 4096

_NC = 2
_NS = 16
_LANES = 16
_NW = _NC * _NS
_BPW = ROWS // _NW
_ICHUNK = 32
_SHARE = ROWS // _NS


def _adj_reduce_body(adj_ref, rdeg_ref, istar_ref):
    b = pl.program_id(1)
    x = adj_ref[0]
    s = jnp.sum(x, axis=1)
    rdeg_ref[0, 0, pl.ds(b * BI, BI)] = lax.rsqrt(s.astype(jnp.float32) + 1.0)
    ii = b * BI + lax.broadcasted_iota(jnp.int32, (BI, N), 0)
    cand = jnp.where(x == 1, ii, -1)
    cm = jnp.max(cand, axis=0, keepdims=True)

    @pl.when(b == 0)
    def _():
        istar_ref[0] = lax.broadcasted_iota(jnp.int32, (1, N), 1)

    istar_ref[0] = jnp.maximum(istar_ref[0], cm)


def _epilogue_body(r_ref, w_ref, s_ref, o_ref):
    t = lax.dot_general(r_ref[...], w_ref[...], (((1,), (1,)), ((), ())),
                        preferred_element_type=jnp.float32)
    o_ref[...] = t * s_ref[...].reshape(BM, 1)


def _sc_gather_scale(istar_hbm, rdeg_hbm, feat_hbm, rows_hbm, scale_hbm,
                     istar_v, rdeg_v, fidx_v, scale_v, rows_v, table_s, sem):
    sid = lax.axis_index("s")
    wid = sid * _NC + lax.axis_index("c")
    base = wid * _BPW

    stage = pltpu.async_copy(feat_hbm.at[pl.ds(sid * _SHARE, _SHARE)],
                             table_s.at[pl.ds(sid * _SHARE, _SHARE)], sem)
    pltpu.sync_copy(istar_hbm, istar_v)
    pltpu.sync_copy(rdeg_hbm, rdeg_v)

    iot = lax.broadcasted_iota(jnp.int32, (_LANES,), 0)
    l_idx = iot & 1
    half = iot >> 1

    def _index_step(k, carry):
        j_idx = ((base >> 1) + 8 * k) + half
        flat_j = l_idx * N + j_idx
        ist = plsc.load_gather(istar_v, [flat_j])
        rs = plsc.load_gather(rdeg_v, [l_idx * N + ist])
        rd = plsc.load_gather(rdeg_v, [flat_j])
        scale_v[pl.ds(_LANES * k, _LANES)] = rs * rd
        fidx_v[pl.ds(_LANES * k, _LANES)] = ist * 2 + l_idx
        return carry

    lax.fori_loop(0, _BPW // _LANES, _index_step, 0)

    stage.wait()
    plsc.subcore_barrier()

    copies = [
        pltpu.async_copy(table_s.at[fidx_v.at[pl.ds(t * _ICHUNK, _ICHUNK)]],
                         rows_v.at[pl.ds(t * _ICHUNK, _ICHUNK)], sem)
        for t in range(_BPW // _ICHUNK)
    ]
    for cp in copies:
        cp.wait()

    pltpu.sync_copy(rows_v, rows_hbm.at[pl.ds(base, _BPW)])
    pltpu.sync_copy(scale_v, scale_hbm.at[pl.ds(base, _BPW)])


@functools.cache
def _sc_kernel():
    mesh = plsc.VectorSubcoreMesh(core_axis_name="c", subcore_axis_name="s",
                                  num_cores=_NC, num_subcores=_NS)
    return pl.kernel(
        _sc_gather_scale,
        out_type=(jax.ShapeDtypeStruct((ROWS, D), jnp.float32),
                  jax.ShapeDtypeStruct((ROWS,), jnp.float32)),
        mesh=mesh,
        scratch_types=[
            pltpu.VMEM((ROWS,), jnp.int32),
            pltpu.VMEM((ROWS,), jnp.float32),
            pltpu.VMEM((_BPW,), jnp.int32),
            pltpu.VMEM((_BPW,), jnp.float32),
            pltpu.VMEM((_BPW, D), jnp.float32),
            pltpu.VMEM_SHARED((ROWS, D), jnp.float32),
            pltpu.SemaphoreType.DMA,
        ],
        compiler_params=pltpu.CompilerParams(needs_layout_passes=False),
    )


def kernel(feature, adj, W):
    rdeg, istar = pl.pallas_call(
        _adj_reduce_body,
        grid=(L, NB),
        in_specs=[pl.BlockSpec((1, BI, N), lambda l, b: (l, b, 0))],
        out_specs=[pl.BlockSpec((1, 1, N), lambda l, b: (l, 0, 0)),
                   pl.BlockSpec((1, 1, N), lambda l, b: (l, 0, 0))],
        out_shape=[jax.ShapeDtypeStruct((L, 1, N), jnp.float32),
                   jax.ShapeDtypeStruct((L, 1, N), jnp.int32)],
    )(adj)

    feat2 = feature.reshape(ROWS, D)
    rows, scale = _sc_kernel()(istar.reshape(ROWS), rdeg.reshape(ROWS), feat2)

    out2 = pl.pallas_call(
        _epilogue_body,
        grid=(ROWS // BM,),
        in_specs=[pl.BlockSpec((BM, D), lambda m: (m, 0)),
                  pl.BlockSpec((D, D), lambda m: (0, 0)),
                  pl.BlockSpec((BM,), lambda m: (m,))],
        out_specs=pl.BlockSpec((BM, D), lambda m: (m, 0)),
        out_shape=jax.ShapeDtypeStruct((ROWS, D), jnp.float32),
    )(rows, W, scale)
    return out2.reshape(N, L, D)


# speedup vs baseline: 2.4370x; 1.0075x over previous
import functools

import jax
import jax.numpy as jnp
from jax import lax
from jax.experimental import pallas as pl
from jax.experimental.pallas import tpu as pltpu
from jax.experimental.pallas import tpu_sc as plsc

N = 4096
L = 2
D = 128
ROWS = N * L
BI = 1024
NB = N // BI
BM = 4096

_NC = 2
_NS = 16
_LANES = 16
_NW = _NC * _NS
_BPW = ROWS // _NW
_ICHUNK = 32
_SHARE = ROWS // _NS


def _adj_reduce_body(adj_ref, rdeg_ref, istar_ref):
    b = pl.program_id(1)
    x = adj_ref[0]
    s = jnp.sum(x, axis=1)
    rdeg_ref[0, 0, pl.ds(b * BI, BI)] = lax.rsqrt(s.astype(jnp.float32) + 1.0)
    ii = b * BI + lax.broadcasted_iota(jnp.int32, (BI, N), 0)
    cm = jnp.max(x * ii, axis=0, keepdims=True)

    @pl.when(b == 0)
    def _():
        istar_ref[0] = lax.broadcasted_iota(jnp.int32, (1, N), 1)

    istar_ref[0] = jnp.maximum(istar_ref[0], cm)


def _epilogue_body(r_ref, w_ref, s_ref, o_ref):
    t = lax.dot_general(r_ref[...], w_ref[...], (((1,), (1,)), ((), ())),
                        preferred_element_type=jnp.float32)
    o_ref[...] = t * s_ref[...].reshape(BM, 1)


def _sc_gather_scale(istar_hbm, rdeg_hbm, feat_hbm, rows_hbm, scale_hbm,
                     istar_v, rdeg_v, fidx_v, scale_v, rows_v, table_s, sem):
    sid = lax.axis_index("s")
    wid = sid * _NC + lax.axis_index("c")
    base = wid * _BPW

    stage = pltpu.async_copy(feat_hbm.at[pl.ds(sid * _SHARE, _SHARE)],
                             table_s.at[pl.ds(sid * _SHARE, _SHARE)], sem)
    pltpu.sync_copy(istar_hbm, istar_v)
    pltpu.sync_copy(rdeg_hbm, rdeg_v)

    iot = lax.broadcasted_iota(jnp.int32, (_LANES,), 0)
    l_idx = iot & 1
    half = iot >> 1

    def _index_step(k, carry):
        j_idx = ((base >> 1) + 8 * k) + half
        flat_j = l_idx * N + j_idx
        ist = plsc.load_gather(istar_v, [flat_j])
        rs = plsc.load_gather(rdeg_v, [l_idx * N + ist])
        rd = plsc.load_gather(rdeg_v, [flat_j])
        scale_v[pl.ds(_LANES * k, _LANES)] = rs * rd
        fidx_v[pl.ds(_LANES * k, _LANES)] = ist * 2 + l_idx
        return carry

    lax.fori_loop(0, _BPW // _LANES, _index_step, 0)

    stage.wait()
    plsc.subcore_barrier()

    copies = [
        pltpu.async_copy(table_s.at[fidx_v.at[pl.ds(t * _ICHUNK, _ICHUNK)]],
                         rows_v.at[pl.ds(t * _ICHUNK, _ICHUNK)], sem)
        for t in range(_BPW // _ICHUNK)
    ]
    for cp in copies:
        cp.wait()

    pltpu.sync_copy(rows_v, rows_hbm.at[pl.ds(base, _BPW)])
    pltpu.sync_copy(scale_v, scale_hbm.at[pl.ds(base, _BPW)])


@functools.cache
def _sc_kernel():
    mesh = plsc.VectorSubcoreMesh(core_axis_name="c", subcore_axis_name="s",
                                  num_cores=_NC, num_subcores=_NS)
    return pl.kernel(
        _sc_gather_scale,
        out_type=(jax.ShapeDtypeStruct((ROWS, D), jnp.float32),
                  jax.ShapeDtypeStruct((ROWS,), jnp.float32)),
        mesh=mesh,
        scratch_types=[
            pltpu.VMEM((ROWS,), jnp.int32),
            pltpu.VMEM((ROWS,), jnp.float32),
            pltpu.VMEM((_BPW,), jnp.int32),
            pltpu.VMEM((_BPW,), jnp.float32),
            pltpu.VMEM((_BPW, D), jnp.float32),
            pltpu.VMEM_SHARED((ROWS, D), jnp.float32),
            pltpu.SemaphoreType.DMA,
        ],
        compiler_params=pltpu.CompilerParams(needs_layout_passes=False),
    )


def kernel(feature, adj, W):
    rdeg, istar = pl.pallas_call(
        _adj_reduce_body,
        grid=(L, NB),
        in_specs=[pl.BlockSpec((1, BI, N), lambda l, b: (l, b, 0))],
        out_specs=[pl.BlockSpec((1, 1, N), lambda l, b: (l, 0, 0)),
                   pl.BlockSpec((1, 1, N), lambda l, b: (l, 0, 0))],
        out_shape=[jax.ShapeDtypeStruct((L, 1, N), jnp.float32),
                   jax.ShapeDtypeStruct((L, 1, N), jnp.int32)],
    )(adj)

    feat2 = feature.reshape(ROWS, D)
    rows, scale = _sc_kernel()(istar.reshape(ROWS), rdeg.reshape(ROWS), feat2)

    out2 = pl.pallas_call(
        _epilogue_body,
        grid=(ROWS // BM,),
        in_specs=[pl.BlockSpec((BM, D), lambda m: (m, 0)),
                  pl.BlockSpec((D, D), lambda m: (0, 0)),
                  pl.BlockSpec((BM,), lambda m: (m,))],
        out_specs=pl.BlockSpec((BM, D), lambda m: (m, 0)),
        out_shape=jax.ShapeDtypeStruct((ROWS, D), jnp.float32),
    )(rows, W, scale)
    return out2.reshape(N, L, D)


# speedup vs baseline: 2.4725x; 1.0146x over previous
import functools

import jax
import jax.numpy as jnp
from jax import lax
from jax.experimental import pallas as pl
from jax.experimental.pallas import tpu as pltpu
from jax.experimental.pallas import tpu_sc as plsc

N = 4096
L = 2
D = 128
ROWS = N * L
BI = 1024
NB = N // BI
BM = 4096

_NC = 2
_NS = 16
_LANES = 16
_NW = _NC * _NS
_BPW = ROWS // _NW
_ICHUNK = 32
_SHARE = ROWS // _NS


def _adj_reduce_body(adj_ref, rdeg_ref, istar_ref):
    b = pl.program_id(1)
    x = adj_ref[0]
    s = jnp.sum(x, axis=1)
    rdeg_ref[0, 0, pl.ds(b * BI, BI)] = lax.rsqrt(s.astype(jnp.float32) + 1.0)
    ii = b * BI + lax.broadcasted_iota(jnp.int32, (BI, N), 0)
    cm = jnp.max(x * ii, axis=0, keepdims=True)

    @pl.when(b == 0)
    def _():
        istar_ref[0] = lax.broadcasted_iota(jnp.int32, (1, N), 1)

    istar_ref[0] = jnp.maximum(istar_ref[0], cm)


def _epilogue_body(r_ref, w_ref, s_ref, o_ref):
    t = lax.dot_general(r_ref[...], w_ref[...], (((1,), (1,)), ((), ())),
                        preferred_element_type=jnp.float32)
    o_ref[...] = t * s_ref[...].reshape(BM, 1)


def _sc_gather_scale(istar_hbm, rdeg_hbm, feat_hbm, rows_hbm, scale_hbm,
                     istar_v, rdeg_v, fidx_v, scale_v, rows_v, mvec_v,
                     masks_rd, table_s, masks_s, sem):
    sid = lax.axis_index("s")
    wid = sid * _NC + lax.axis_index("c")
    base = wid * _BPW

    pltpu.sync_copy(istar_hbm, istar_v)
    pltpu.sync_copy(rdeg_hbm, rdeg_v)

    iot = lax.broadcasted_iota(jnp.int32, (_LANES,), 0)
    l_idx = iot & 1
    half = iot >> 1

    def _index_step(k, maskvec):
        j_idx = ((base >> 1) + 8 * k) + half
        flat_j = l_idx * N + j_idx
        ist = plsc.load_gather(istar_v, [flat_j])
        rs = plsc.load_gather(rdeg_v, [l_idx * N + ist])
        rd = plsc.load_gather(rdeg_v, [flat_j])
        scale_v[pl.ds(_LANES * k, _LANES)] = rs * rd
        fidx = ist * 2 + l_idx
        fidx_v[pl.ds(_LANES * k, _LANES)] = fidx
        return maskvec | (1 << (fidx >> 9))

    maskvec = lax.fori_loop(0, _BPW // _LANES, _index_step,
                            jnp.zeros((_LANES,), jnp.int32))

    mvec_v[...] = maskvec
    pltpu.sync_copy(mvec_v, masks_s.at[pl.ds(sid * _LANES, _LANES)])
    plsc.subcore_barrier()
    pltpu.sync_copy(masks_s, masks_rd)
    acc = masks_rd[pl.ds(0, _LANES)]
    for k in range(1, _NS):
        acc = acc | masks_rd[pl.ds(k * _LANES, _LANES)]
    need = lax.reduce_max((acc >> sid) & 1, (0,))

    @pl.when(need > 0)
    def _():
        pltpu.sync_copy(feat_hbm.at[pl.ds(sid * _SHARE, _SHARE)],
                        table_s.at[pl.ds(sid * _SHARE, _SHARE)])

    plsc.subcore_barrier()

    copies = [
        pltpu.async_copy(table_s.at[fidx_v.at[pl.ds(t * _ICHUNK, _ICHUNK)]],
                         rows_v.at[pl.ds(t * _ICHUNK, _ICHUNK)], sem)
        for t in range(_BPW // _ICHUNK)
    ]
    for cp in copies:
        cp.wait()

    pltpu.sync_copy(rows_v, rows_hbm.at[pl.ds(base, _BPW)])
    pltpu.sync_copy(scale_v, scale_hbm.at[pl.ds(base, _BPW)])


@functools.cache
def _sc_kernel():
    mesh = plsc.VectorSubcoreMesh(core_axis_name="c", subcore_axis_name="s",
                                  num_cores=_NC, num_subcores=_NS)
    return pl.kernel(
        _sc_gather_scale,
        out_type=(jax.ShapeDtypeStruct((ROWS, D), jnp.float32),
                  jax.ShapeDtypeStruct((ROWS,), jnp.float32)),
        mesh=mesh,
        scratch_types=[
            pltpu.VMEM((ROWS,), jnp.int32),
            pltpu.VMEM((ROWS,), jnp.float32),
            pltpu.VMEM((_BPW,), jnp.int32),
            pltpu.VMEM((_BPW,), jnp.float32),
            pltpu.VMEM((_BPW, D), jnp.float32),
            pltpu.VMEM((_LANES,), jnp.int32),
            pltpu.VMEM((_NS * _LANES,), jnp.int32),
            pltpu.VMEM_SHARED((ROWS, D), jnp.float32),
            pltpu.VMEM_SHARED((_NS * _LANES,), jnp.int32),
            pltpu.SemaphoreType.DMA,
        ],
        compiler_params=pltpu.CompilerParams(needs_layout_passes=False),
    )


def kernel(feature, adj, W):
    rdeg, istar = pl.pallas_call(
        _adj_reduce_body,
        grid=(L, NB),
        in_specs=[pl.BlockSpec((1, BI, N), lambda l, b: (l, b, 0))],
        out_specs=[pl.BlockSpec((1, 1, N), lambda l, b: (l, 0, 0)),
                   pl.BlockSpec((1, 1, N), lambda l, b: (l, 0, 0))],
        out_shape=[jax.ShapeDtypeStruct((L, 1, N), jnp.float32),
                   jax.ShapeDtypeStruct((L, 1, N), jnp.int32)],
    )(adj)

    feat2 = feature.reshape(ROWS, D)
    rows, scale = _sc_kernel()(istar.reshape(ROWS), rdeg.reshape(ROWS), feat2)

    out2 = pl.pallas_call(
        _epilogue_body,
        grid=(ROWS // BM,),
        in_specs=[pl.BlockSpec((BM, D), lambda m: (m, 0)),
                  pl.BlockSpec((D, D), lambda m: (0, 0)),
                  pl.BlockSpec((BM,), lambda m: (m,))],
        out_specs=pl.BlockSpec((BM, D), lambda m: (m, 0)),
        out_shape=jax.ShapeDtypeStruct((ROWS, D), jnp.float32),
    )(rows, W, scale)
    return out2.reshape(N, L, D)


# speedup vs baseline: 2.4743x; 1.0007x over previous
import functools

import jax
import jax.numpy as jnp
from jax import lax
from jax.experimental import pallas as pl
from jax.experimental.pallas import tpu as pltpu
from jax.experimental.pallas import tpu_sc as plsc

N = 4096
L = 2
D = 128
ROWS = N * L
BI = 1024
NB = N // BI
BM = 4096

_NC = 2
_NS = 16
_LANES = 16
_NW = _NC * _NS
_BPW = ROWS // _NW
_ICHUNK = 64
_SHARE = ROWS // _NS


def _adj_reduce_body(adj_ref, rdeg_ref, istar_ref):
    b = pl.program_id(1)
    x = adj_ref[0]
    s = jnp.sum(x, axis=1)
    rdeg_ref[0, 0, pl.ds(b * BI, BI)] = lax.rsqrt(s.astype(jnp.float32) + 1.0)
    ii = b * BI + lax.broadcasted_iota(jnp.int32, (BI, N), 0)
    cm = jnp.max(x * ii, axis=0, keepdims=True)

    @pl.when(b == 0)
    def _():
        istar_ref[0] = lax.broadcasted_iota(jnp.int32, (1, N), 1)

    istar_ref[0] = jnp.maximum(istar_ref[0], cm)


def _epilogue_body(r_ref, w_ref, s_ref, o_ref):
    t = lax.dot_general(r_ref[...], w_ref[...], (((1,), (1,)), ((), ())),
                        preferred_element_type=jnp.float32)
    o_ref[...] = t * s_ref[...].reshape(BM, 1)


def _sc_gather_scale(istar_hbm, rdeg_hbm, feat_hbm, rows_hbm, scale_hbm,
                     istar_v, rdeg_v, fidx_v, scale_v, rows_v, mvec_v,
                     masks_rd, table_s, masks_s, sem):
    sid = lax.axis_index("s")
    wid = sid * _NC + lax.axis_index("c")
    base = wid * _BPW

    pltpu.sync_copy(istar_hbm, istar_v)
    pltpu.sync_copy(rdeg_hbm, rdeg_v)

    iot = lax.broadcasted_iota(jnp.int32, (_LANES,), 0)
    l_idx = iot & 1
    half = iot >> 1

    def _index_step(k, maskvec):
        j_idx = ((base >> 1) + 8 * k) + half
        flat_j = l_idx * N + j_idx
        ist = plsc.load_gather(istar_v, [flat_j])
        rs = plsc.load_gather(rdeg_v, [l_idx * N + ist])
        rd = plsc.load_gather(rdeg_v, [flat_j])
        scale_v[pl.ds(_LANES * k, _LANES)] = rs * rd
        fidx = ist * 2 + l_idx
        fidx_v[pl.ds(_LANES * k, _LANES)] = fidx
        return maskvec | (1 << (fidx >> 9))

    maskvec = lax.fori_loop(0, _BPW // _LANES, _index_step,
                            jnp.zeros((_LANES,), jnp.int32))

    mvec_v[...] = maskvec
    pltpu.sync_copy(mvec_v, masks_s.at[pl.ds(sid * _LANES, _LANES)])
    plsc.subcore_barrier()
    pltpu.sync_copy(masks_s, masks_rd)
    acc = masks_rd[pl.ds(0, _LANES)]
    for k in range(1, _NS):
        acc = acc | masks_rd[pl.ds(k * _LANES, _LANES)]
    need = lax.reduce_max((acc >> sid) & 1, (0,))

    @pl.when(need > 0)
    def _():
        pltpu.sync_copy(feat_hbm.at[pl.ds(sid * _SHARE, _SHARE)],
                        table_s.at[pl.ds(sid * _SHARE, _SHARE)])

    plsc.subcore_barrier()

    copies = [
        pltpu.async_copy(table_s.at[fidx_v.at[pl.ds(t * _ICHUNK, _ICHUNK)]],
                         rows_v.at[pl.ds(t * _ICHUNK, _ICHUNK)], sem)
        for t in range(_BPW // _ICHUNK)
    ]
    for cp in copies:
        cp.wait()

    pltpu.sync_copy(rows_v, rows_hbm.at[pl.ds(base, _BPW)])
    pltpu.sync_copy(scale_v, scale_hbm.at[pl.ds(base, _BPW)])


@functools.cache
def _sc_kernel():
    mesh = plsc.VectorSubcoreMesh(core_axis_name="c", subcore_axis_name="s",
                                  num_cores=_NC, num_subcores=_NS)
    return pl.kernel(
        _sc_gather_scale,
        out_type=(jax.ShapeDtypeStruct((ROWS, D), jnp.float32),
                  jax.ShapeDtypeStruct((ROWS,), jnp.float32)),
        mesh=mesh,
        scratch_types=[
            pltpu.VMEM((ROWS,), jnp.int32),
            pltpu.VMEM((ROWS,), jnp.float32),
            pltpu.VMEM((_BPW,), jnp.int32),
            pltpu.VMEM((_BPW,), jnp.float32),
            pltpu.VMEM((_BPW, D), jnp.float32),
            pltpu.VMEM((_LANES,), jnp.int32),
            pltpu.VMEM((_NS * _LANES,), jnp.int32),
            pltpu.VMEM_SHARED((ROWS, D), jnp.float32),
            pltpu.VMEM_SHARED((_NS * _LANES,), jnp.int32),
            pltpu.SemaphoreType.DMA,
        ],
        compiler_params=pltpu.CompilerParams(needs_layout_passes=False),
    )


def kernel(feature, adj, W):
    rdeg, istar = pl.pallas_call(
        _adj_reduce_body,
        grid=(L, NB),
        in_specs=[pl.BlockSpec((1, BI, N), lambda l, b: (l, b, 0))],
        out_specs=[pl.BlockSpec((1, 1, N), lambda l, b: (l, 0, 0)),
                   pl.BlockSpec((1, 1, N), lambda l, b: (l, 0, 0))],
        out_shape=[jax.ShapeDtypeStruct((L, 1, N), jnp.float32),
                   jax.ShapeDtypeStruct((L, 1, N), jnp.int32)],
    )(adj)

    feat2 = feature.reshape(ROWS, D)
    rows, scale = _sc_kernel()(istar.reshape(ROWS), rdeg.reshape(ROWS), feat2)

    out2 = pl.pallas_call(
        _epilogue_body,
        grid=(ROWS // BM,),
        in_specs=[pl.BlockSpec((BM, D), lambda m: (m, 0)),
                  pl.BlockSpec((D, D), lambda m: (0, 0)),
                  pl.BlockSpec((BM,), lambda m: (m,))],
        out_specs=pl.BlockSpec((BM, D), lambda m: (m, 0)),
        out_shape=jax.ShapeDtypeStruct((ROWS, D), jnp.float32),
    )(rows, W, scale)
    return out2.reshape(N, L, D)
